# baseline (speedup 1.0000x reference)
import functools

import jax
import jax.numpy as jnp
from jax import lax
from jax.experimental import pallas as pl
from jax.experimental.pallas import tpu as pltpu
from jax.experimental.pallas import tpu_sc as plsc

N = 10000
NP = 10112
D = 128
E = 640000

NC = 2
NS = 16
E_BLK = 80
E_TILE = 20480
EP = NC * NS * E_TILE
N_CHUNK = E_TILE // E_BLK
PERIOD = 8
N_PERIOD = N_CHUNK // PERIOD
ROWS_TILE = NP // NS

_mesh = plsc.VectorSubcoreMesh(core_axis_name="c", subcore_axis_name="s")


@functools.partial(
    pl.kernel,
    out_type=jax.ShapeDtypeStruct((NC * NP, D), jnp.float32),
    mesh=_mesh,
    scratch_types=[
        pltpu.VMEM_SHARED((NP, D), jnp.float32),
        pltpu.VMEM((2 * PERIOD, E_BLK), jnp.int32),
        pltpu.VMEM((2 * PERIOD, E_BLK), jnp.int32),
        pltpu.VMEM((2 * PERIOD, E_BLK), jnp.float32),
        [pltpu.VMEM((E_BLK, D), jnp.float32)] * 4,
        pltpu.SemaphoreType.DMA,
        [pltpu.SemaphoreType.DMA] * 4,
        [pltpu.SemaphoreType.DMA] * 4,
    ],
)
def _spmm_layer(x_hbm, cols_hbm, rows_hbm, vals_hbm, zeros_hbm, out_hbm,
                acc, cbuf, rbuf, vbuf, gbuf, hsem, gsem, ssem):
    c = lax.axis_index("c")
    s = lax.axis_index("s")

    pltpu.sync_copy(zeros_hbm, acc.at[pl.ds(s * ROWS_TILE, ROWS_TILE)])
    plsc.subcore_barrier()

    cbc = (c * NS + s) * N_CHUNK

    def stage_period(p):
        src = pl.ds(cbc + p * PERIOD, PERIOD)
        hrow = lax.rem(p, 2) * PERIOD
        dst = pl.ds(hrow, PERIOD)
        pltpu.async_copy(cols_hbm.at[src], cbuf.at[dst], hsem)
        pltpu.async_copy(rows_hbm.at[src], rbuf.at[dst], hsem)
        pltpu.async_copy(vals_hbm.at[src], vbuf.at[dst], hsem)

    def wait_period(p):
        src = pl.ds(cbc + p * PERIOD, PERIOD)
        hrow = lax.rem(p, 2) * PERIOD
        dst = pl.ds(hrow, PERIOD)
        pltpu.make_async_copy(cols_hbm.at[src], cbuf.at[dst], hsem).wait()
        pltpu.make_async_copy(rows_hbm.at[src], rbuf.at[dst], hsem).wait()
        pltpu.make_async_copy(vals_hbm.at[src], vbuf.at[dst], hsem).wait()

    def start_gather(jj, i):
        pltpu.async_copy(x_hbm.at[cbuf.at[jj]], gbuf[i], gsem[i])

    def wait_gather(jj, i):
        pltpu.make_async_copy(x_hbm.at[cbuf.at[jj]], gbuf[i],
                              gsem[i]).wait()

    def start_scatter(jj, i):
        pltpu.async_copy(gbuf[i], acc.at[rbuf.at[jj]], ssem[i], add=True)

    def wait_scatter(jj, i):
        pltpu.make_async_copy(gbuf[i], acc.at[rbuf.at[jj]], ssem[i]).wait()

    def chunk_body(jj0, i, wait_prev_sc, do_gather):
        i2 = (i + 2) % 4
        jj = jj0 + i
        wait_gather(jj, i)

        def scale_group(g, carry2):
            v16 = vbuf[jj, pl.ds(g * 16, 16)]
            for j in range(16):
                e = g * 16 + j
                v = v16[j]
                for d in range(D // 16):
                    sl = pl.ds(d * 16, 16)
                    gbuf[i][e, sl] = gbuf[i][e, sl] * v
            return carry2

        lax.fori_loop(0, E_BLK // 16, scale_group, 0)
        start_scatter(jj, i)
        if wait_prev_sc:
            wait_scatter(lax.rem(jj + 14, 16), i2)
        if do_gather:
            start_gather(lax.rem(jj + 2, 16), i2)

    stage_period(0)
    stage_period(1)
    wait_period(0)
    start_gather(0, 0)
    start_gather(1, 1)

    def quad_body(q, carry):
        qm = lax.rem(q, 4)
        jj0 = qm * 4
        p = q // 2
        for i in range(4):
            if i == 2:

                @pl.when(jnp.logical_and(lax.rem(q, 2) == 0,
                                         jnp.logical_and(q >= 2, q <= 60)))
                def _():
                    stage_period(p + 1)

                @pl.when(jnp.logical_and(lax.rem(q, 2) == 1, q <= 61))
                def _():
                    wait_period(p + 1)

            chunk_body(jj0, i, True, True)
        return carry

    def quad_static(q, wait_prev_sc_first2, do_gather_last2):
        jj0 = (q % 4) * 4
        chunk_body(jj0, 0, wait_prev_sc_first2, True)
        chunk_body(jj0, 1, wait_prev_sc_first2, True)
        chunk_body(jj0, 2, True, do_gather_last2)
        chunk_body(jj0, 3, True, do_gather_last2)

    quad_static(0, False, True)
    lax.fori_loop(1, N_CHUNK // 4 - 1, quad_body, 0)
    quad_static(N_CHUNK // 4 - 1, True, False)
    wait_scatter(14, 2)
    wait_scatter(15, 3)
    plsc.subcore_barrier()

    row0 = s * ROWS_TILE
    pltpu.sync_copy(acc.at[pl.ds(row0, ROWS_TILE)],
                    out_hbm.at[pl.ds(c * NP + row0, ROWS_TILE)])


_BLK = 1000
_BLKP = 632


def _add2_body(a_ref, b_ref, o_ref):
    o_ref[...] = a_ref[...] + b_ref[...]


def _combine_partials(p):
    return pl.pallas_call(
        _add2_body,
        out_shape=jax.ShapeDtypeStruct((NP, D), jnp.float32),
        grid=(NP // _BLKP,),
        in_specs=[
            pl.BlockSpec((_BLKP, D), lambda i: (i, 0)),
            pl.BlockSpec((_BLKP, D), lambda i: (i + NP // _BLKP, 0)),
        ],
        out_specs=pl.BlockSpec((_BLKP, D), lambda i: (i, 0)),
    )(p, p)


def _mean_body(x0_ref, x1_ref, a_ref, b_ref, o_ref):
    o_ref[...] = (x0_ref[...] + x1_ref[...] + a_ref[...] + b_ref[...]) * (1.0 / 3.0)


def _final_mean(x0, x1, p2a, p2b):
    return pl.pallas_call(
        _mean_body,
        out_shape=jax.ShapeDtypeStruct((N, D), jnp.float32),
        grid=(N // _BLK,),
        in_specs=[pl.BlockSpec((_BLK, D), lambda i: (i, 0))] * 4,
        out_specs=pl.BlockSpec((_BLK, D), lambda i: (i, 0)),
    )(x0, x1, p2a, p2b)


def kernel(adj1_indices, adj1_values, adj2_indices, adj2_values, user_emb, item_emb):
    pad_i = jnp.arange(EP - E, dtype=jnp.int32) % N
    rows = jnp.concatenate(
        [adj1_indices[0], adj2_indices[0], pad_i]).reshape(-1, E_BLK)
    cols = jnp.concatenate(
        [adj1_indices[1], adj2_indices[1], pad_i]).reshape(-1, E_BLK)
    vals = jnp.concatenate(
        [adj1_values, adj2_values,
         jnp.zeros((EP - E,), jnp.float32)]).reshape(-1, E_BLK)
    x0 = jnp.concatenate([item_emb, user_emb], axis=0)
    zeros = jnp.zeros((ROWS_TILE, D), jnp.float32)

    p1 = _spmm_layer(x0, cols, rows, vals, zeros)
    x1 = _combine_partials(p1)
    p2 = _spmm_layer(x1, cols, rows, vals, zeros)
    return _final_mean(x0, x1[:N], p2[:N], p2[NP:NP + N])


# speedup vs baseline: 1.0550x; 1.0550x over previous
import functools

import jax
import jax.numpy as jnp
from jax import lax
from jax.experimental import pallas as pl
from jax.experimental.pallas import tpu as pltpu
from jax.experimental.pallas import tpu_sc as plsc

N = 10000
NP = 10240
D = 128
E = 640000

NC = 2
NS = 16
E_BLK = 80
E_TILE = 20160
EP = NC * NS * E_TILE
N_CHUNK = E_TILE // E_BLK
ROWS_TILE = NP // NS

_mesh = plsc.VectorSubcoreMesh(core_axis_name="c", subcore_axis_name="s")


@functools.partial(
    pl.kernel,
    out_type=jax.ShapeDtypeStruct((NC * NP, D), jnp.float32),
    mesh=_mesh,
    scratch_types=[
        pltpu.VMEM_SHARED((NP, D), jnp.float32),
        [pltpu.VMEM((E_BLK,), jnp.int32)] * 4,
        [pltpu.VMEM((E_BLK,), jnp.int32)] * 4,
        [pltpu.VMEM((E_BLK,), jnp.float32)] * 4,
        [pltpu.VMEM((E_BLK, D), jnp.float32)] * 4,
        [pltpu.SemaphoreType.DMA] * 4,
        [pltpu.SemaphoreType.DMA] * 4,
        [pltpu.SemaphoreType.DMA] * 4,
        [pltpu.SemaphoreType.DMA] * 4,
    ],
)
def _spmm_layer(x_hbm, cols_hbm, rows_hbm, vals_hbm, zeros_hbm, out_hbm,
                acc, cbuf, rbuf, vbuf, gbuf, csem, rsem, gsem, ssem):
    c = lax.axis_index("c")
    s = lax.axis_index("s")

    cb = (c * NS + s) * E_TILE

    def eslice(k):
        return pl.ds(cb + k * E_BLK, E_BLK)

    def stage_c(k, i):
        pltpu.async_copy(cols_hbm.at[eslice(k)], cbuf[i], csem[i])

    def wait_stage_c(k, i):
        pltpu.make_async_copy(cols_hbm.at[eslice(k)], cbuf[i],
                              csem[i]).wait()

    def stage_rv(k, i):
        pltpu.async_copy(rows_hbm.at[eslice(k)], rbuf[i], rsem[i])
        pltpu.async_copy(vals_hbm.at[eslice(k)], vbuf[i], rsem[i])

    def wait_stage_rv(k, i):
        pltpu.make_async_copy(rows_hbm.at[eslice(k)], rbuf[i],
                              rsem[i]).wait()
        pltpu.make_async_copy(vals_hbm.at[eslice(k)], vbuf[i],
                              rsem[i]).wait()

    def start_gather(i):
        pltpu.async_copy(x_hbm.at[cbuf[i]], gbuf[i], gsem[i])

    def wait_gather(i):
        pltpu.make_async_copy(x_hbm.at[cbuf[i]], gbuf[i], gsem[i]).wait()

    def start_scatter(i):
        pltpu.async_copy(gbuf[i], acc.at[rbuf[i]], ssem[i], add=True)

    def wait_scatter(i):
        pltpu.make_async_copy(gbuf[i], acc.at[rbuf[i]], ssem[i]).wait()

    def chunk_body(k, i, wait_prev_sc, do_stage_c, do_stage_rv, do_gather):
        i2 = (i + 2) % 4
        wait_stage_rv(k, i)
        wait_gather(i)

        def scale_group(g, carry2):
            v16 = vbuf[i][pl.ds(g * 16, 16)]
            for j in range(16):
                e = g * 16 + j
                v = v16[j]
                for d in range(D // 16):
                    sl = pl.ds(d * 16, 16)
                    gbuf[i][e, sl] = gbuf[i][e, sl] * v
            return carry2

        lax.fori_loop(0, E_BLK // 16, scale_group, 0)
        start_scatter(i)
        if do_stage_c:
            stage_c(k + 4, i)
        if wait_prev_sc:
            wait_scatter(i2)
        if do_stage_rv:
            stage_rv(k + 2, i2)
        if do_gather:
            wait_stage_c(k + 2, i2)
            start_gather(i2)

    for i in range(4):
        stage_c(i, i)
        stage_rv(i, i)
    pltpu.sync_copy(zeros_hbm, acc.at[pl.ds(s * ROWS_TILE, ROWS_TILE)])
    plsc.subcore_barrier()
    wait_stage_c(0, 0)
    start_gather(0)
    wait_stage_c(1, 1)
    start_gather(1)

    chunk_body(0, 0, False, True, False, True)
    chunk_body(1, 1, False, True, False, True)
    chunk_body(2, 2, True, True, True, True)
    chunk_body(3, 3, True, True, True, True)

    def quad_body(q, carry):
        k0 = 4 * q
        for i in range(4):
            chunk_body(k0 + i, i, True, True, True, True)
        return carry

    lax.fori_loop(1, N_CHUNK // 4 - 1, quad_body, 0)

    kl = N_CHUNK - 4
    chunk_body(kl + 0, 0, True, False, True, True)
    chunk_body(kl + 1, 1, True, False, True, True)
    chunk_body(kl + 2, 2, True, False, False, False)
    chunk_body(kl + 3, 3, True, False, False, False)
    wait_scatter(2)
    wait_scatter(3)
    plsc.subcore_barrier()

    row0 = s * ROWS_TILE
    pltpu.sync_copy(acc.at[pl.ds(row0, ROWS_TILE)],
                    out_hbm.at[pl.ds(c * NP + row0, ROWS_TILE)])


_BLK = 1000
_BLKP = 1024


def _add2_body(a_ref, b_ref, o_ref):
    o_ref[...] = a_ref[...] + b_ref[...]


def _combine_partials(p):
    return pl.pallas_call(
        _add2_body,
        out_shape=jax.ShapeDtypeStruct((NP, D), jnp.float32),
        grid=(NP // _BLKP,),
        in_specs=[
            pl.BlockSpec((_BLKP, D), lambda i: (i, 0)),
            pl.BlockSpec((_BLKP, D), lambda i: (i + NP // _BLKP, 0)),
        ],
        out_specs=pl.BlockSpec((_BLKP, D), lambda i: (i, 0)),
    )(p, p)


def _mean_body(x0_ref, x1_ref, a_ref, b_ref, o_ref):
    o_ref[...] = (x0_ref[...] + x1_ref[...] + a_ref[...] + b_ref[...]) * (1.0 / 3.0)


def _final_mean(x0, x1, p2a, p2b):
    return pl.pallas_call(
        _mean_body,
        out_shape=jax.ShapeDtypeStruct((N, D), jnp.float32),
        grid=(N // _BLK,),
        in_specs=[pl.BlockSpec((_BLK, D), lambda i: (i, 0))] * 4,
        out_specs=pl.BlockSpec((_BLK, D), lambda i: (i, 0)),
    )(x0, x1, p2a, p2b)


def kernel(adj1_indices, adj1_values, adj2_indices, adj2_values, user_emb, item_emb):
    pad_i = jnp.arange(EP - E, dtype=jnp.int32) % N
    rows = jnp.concatenate([adj1_indices[0], adj2_indices[0], pad_i])
    cols = jnp.concatenate([adj1_indices[1], adj2_indices[1], pad_i])
    vals = jnp.concatenate(
        [adj1_values, adj2_values, jnp.zeros((EP - E,), jnp.float32)])
    x0 = jnp.concatenate([item_emb, user_emb], axis=0)
    zeros = jnp.zeros((ROWS_TILE, D), jnp.float32)

    p1 = _spmm_layer(x0, cols, rows, vals, zeros)
    x1 = _combine_partials(p1)
    p2 = _spmm_layer(x1, cols, rows, vals, zeros)
    return _final_mean(x0, x1[:N], p2[:N], p2[NP:NP + N])


# speedup vs baseline: 1.0766x; 1.0205x over previous
import functools

import jax
import jax.numpy as jnp
from jax import lax
from jax.experimental import pallas as pl
from jax.experimental.pallas import tpu as pltpu
from jax.experimental.pallas import tpu_sc as plsc

N = 10000
NP = 10240
D = 128
E = 640000

NC = 2
NS = 16
E_BLK = 80
E_TILE = 20160
EP = NC * NS * E_TILE
N_CHUNK = E_TILE // E_BLK
ROWS_TILE = NP // NS

_mesh = plsc.VectorSubcoreMesh(core_axis_name="c", subcore_axis_name="s")


@functools.partial(
    pl.kernel,
    out_type=jax.ShapeDtypeStruct((NC * NP, D), jnp.float32),
    mesh=_mesh,
    scratch_types=[
        pltpu.VMEM_SHARED((NP, D), jnp.float32),
        [pltpu.VMEM((E_BLK,), jnp.int32)] * 4,
        [pltpu.VMEM((E_BLK,), jnp.int32)] * 4,
        [pltpu.VMEM((E_BLK,), jnp.float32)] * 4,
        [pltpu.VMEM((E_BLK, D), jnp.float32)] * 4,
        [pltpu.SemaphoreType.DMA] * 4,
        [pltpu.SemaphoreType.DMA] * 4,
        [pltpu.SemaphoreType.DMA] * 4,
        [pltpu.SemaphoreType.DMA] * 4,
    ],
)
def _spmm_layer(x_hbm, cols_hbm, rows_hbm, vals_hbm, zeros_hbm, out_hbm,
                acc, cbuf, rbuf, vbuf, gbuf, csem, rsem, gsem, ssem):
    c = lax.axis_index("c")
    s = lax.axis_index("s")

    cb = (c * NS + s) * E_TILE

    def eslice(k):
        return pl.ds(cb + k * E_BLK, E_BLK)

    def stage_c(k, i):
        pltpu.async_copy(cols_hbm.at[eslice(k)], cbuf[i], csem[i])

    def wait_stage_c(k, i):
        pltpu.make_async_copy(cols_hbm.at[eslice(k)], cbuf[i],
                              csem[i]).wait()

    def stage_rv(k, i):
        pltpu.async_copy(rows_hbm.at[eslice(k)], rbuf[i], rsem[i])
        pltpu.async_copy(vals_hbm.at[eslice(k)], vbuf[i], rsem[i])

    def wait_stage_rv(k, i):
        pltpu.make_async_copy(rows_hbm.at[eslice(k)], rbuf[i],
                              rsem[i]).wait()
        pltpu.make_async_copy(vals_hbm.at[eslice(k)], vbuf[i],
                              rsem[i]).wait()

    def start_gather(i):
        pltpu.async_copy(x_hbm.at[cbuf[i]], gbuf[i], gsem[i])

    def wait_gather(i):
        pltpu.make_async_copy(x_hbm.at[cbuf[i]], gbuf[i], gsem[i]).wait()

    def start_scatter(i):
        pltpu.async_copy(gbuf[i], acc.at[rbuf[i]], ssem[i], add=True)

    def wait_scatter(i):
        pltpu.make_async_copy(gbuf[i], acc.at[rbuf[i]], ssem[i]).wait()

    def chunk_body(k, i, wait_prev_sc, do_stage_c, do_stage_rv, do_gather):
        i2 = (i + 2) % 4
        wait_stage_rv(k, i)
        wait_gather(i)

        def scale_group(g, carry2):
            v16 = vbuf[i][pl.ds(g * 16, 16)]
            for j in range(16):
                e = g * 16 + j
                v = v16[j]
                for d in range(D // 16):
                    sl = pl.ds(d * 16, 16)
                    gbuf[i][e, sl] = gbuf[i][e, sl] * v
            return carry2

        lax.fori_loop(0, E_BLK // 16, scale_group, 0)
        start_scatter(i)
        if do_stage_c:
            stage_c(k + 4, i)
        if wait_prev_sc:
            wait_scatter(i2)
        if do_stage_rv:
            stage_rv(k + 2, i2)
        if do_gather:
            wait_stage_c(k + 2, i2)
            start_gather(i2)

    for i in range(4):
        stage_c(i, i)
        stage_rv(i, i)
    pltpu.sync_copy(zeros_hbm, acc.at[pl.ds(s * ROWS_TILE, ROWS_TILE)])
    plsc.subcore_barrier()
    wait_stage_c(0, 0)
    start_gather(0)
    wait_stage_c(1, 1)
    start_gather(1)

    chunk_body(0, 0, False, True, False, True)
    chunk_body(1, 1, False, True, False, True)
    chunk_body(2, 2, True, True, True, True)
    chunk_body(3, 3, True, True, True, True)

    def quad_body(q, carry):
        k0 = 4 * q
        for i in range(4):
            chunk_body(k0 + i, i, True, True, True, True)
        return carry

    lax.fori_loop(1, N_CHUNK // 4 - 1, quad_body, 0)

    kl = N_CHUNK - 4
    chunk_body(kl + 0, 0, True, False, True, True)
    chunk_body(kl + 1, 1, True, False, True, True)
    chunk_body(kl + 2, 2, True, False, False, False)
    chunk_body(kl + 3, 3, True, False, False, False)
    wait_scatter(2)
    wait_scatter(3)
    plsc.subcore_barrier()

    row0 = s * ROWS_TILE
    pltpu.sync_copy(acc.at[pl.ds(row0, ROWS_TILE)],
                    out_hbm.at[pl.ds(c * NP + row0, ROWS_TILE)])


_BLK = 1000
_BLKP = 1024


def _add2_body(a_ref, b_ref, o_ref):
    o_ref[...] = a_ref[...] + b_ref[...]


def _combine_partials(p):
    return pl.pallas_call(
        _add2_body,
        out_shape=jax.ShapeDtypeStruct((NP, D), jnp.float32),
        grid=(NP // _BLKP,),
        in_specs=[
            pl.BlockSpec((_BLKP, D), lambda i: (i, 0)),
            pl.BlockSpec((_BLKP, D), lambda i: (i + NP // _BLKP, 0)),
        ],
        out_specs=pl.BlockSpec((_BLKP, D), lambda i: (i, 0)),
    )(p, p)


def _mean_body(x0_ref, x1_ref, a_ref, b_ref, o_ref):
    o_ref[...] = (x0_ref[...] + x1_ref[...] + a_ref[0] + b_ref[0]) * (1.0 / 3.0)


def _final_mean(x0, x1, p2):
    p3 = p2.reshape(2, NP, D)
    return pl.pallas_call(
        _mean_body,
        out_shape=jax.ShapeDtypeStruct((N, D), jnp.float32),
        grid=(N // _BLK,),
        in_specs=[
            pl.BlockSpec((_BLK, D), lambda i: (i, 0)),
            pl.BlockSpec((_BLK, D), lambda i: (i, 0)),
            pl.BlockSpec((1, _BLK, D), lambda i: (0, i, 0)),
            pl.BlockSpec((1, _BLK, D), lambda i: (1, i, 0)),
        ],
        out_specs=pl.BlockSpec((_BLK, D), lambda i: (i, 0)),
    )(x0, x1, p3, p3)


def kernel(adj1_indices, adj1_values, adj2_indices, adj2_values, user_emb, item_emb):
    pad_i = jnp.arange(EP - E, dtype=jnp.int32) % N
    rows = jnp.concatenate([adj1_indices[0], adj2_indices[0], pad_i])
    cols = jnp.concatenate([adj1_indices[1], adj2_indices[1], pad_i])
    vals = jnp.concatenate(
        [adj1_values, adj2_values, jnp.zeros((EP - E,), jnp.float32)])
    x0 = jnp.concatenate([item_emb, user_emb], axis=0)
    zeros = jnp.zeros((ROWS_TILE, D), jnp.float32)

    p1 = _spmm_layer(x0, cols, rows, vals, zeros)
    x1 = _combine_partials(p1)
    p2 = _spmm_layer(x1, cols, rows, vals, zeros)
    return _final_mean(x0, x1, p2)
